# Initial kernel scaffold; baseline (speedup 1.0000x reference)
import jax
import jax.numpy as jnp
from jax.experimental import pallas as pl


def kernel(edges, graph_embedding, W1, b1, W2, b2):
    raise NotImplementedError("write your pallas kernel here")


# speedup vs baseline: 6.6833x; 6.6833x over previous
import functools

import jax
import jax.numpy as jnp
from jax import lax
from jax.experimental import pallas as pl
from jax.experimental.pallas import tpu as pltpu
from jax.experimental.pallas import tpu_sc as plsc

_N = 10000
_D = 128
_E = 320000
_NP = 10240

_NW = 32
_CH = 159
_C = _CH * 128
_TOT = _NW * _C

_sc_mesh = plsc.VectorSubcoreMesh(core_axis_name="c", subcore_axis_name="s")


@functools.partial(
    pl.kernel,
    out_type=(),
    mesh=_sc_mesh,
    scratch_types=[
        pltpu.VMEM((_CH, 128), jnp.int32),
        pltpu.VMEM((_CH, 128), jnp.int32),
        pltpu.VMEM((_CH, 128), jnp.int32),
        pltpu.VMEM((128,), jnp.float32),
        pltpu.SemaphoreType.DMA,
    ],
)
def _scatter_adj(src_hbm, dst_hbm, a_hbm, src_v, dst_v, idx_v, ones_v, sem):
    w = lax.axis_index("s") * 2 + lax.axis_index("c")
    pltpu.sync_copy(src_hbm.at[w], src_v)
    pltpu.sync_copy(dst_hbm.at[w], dst_v)
    for k in range(8):
        ones_v[pl.ds(16 * k, 16)] = jnp.ones((16,), jnp.float32)

    def jbody(j, carry):
        for k in range(8):
            sl = pl.ds(16 * k, 16)
            idx_v[j, sl] = src_v[j, sl] * _NP + dst_v[j, sl]
        return carry

    lax.fori_loop(0, _CH, jbody, 0)

    def fire(j, carry):
        pltpu.async_copy(ones_v, a_hbm.at[idx_v.at[j]], sem)
        return carry

    lax.fori_loop(0, _CH, fire, 0)

    def drain(j, carry):
        pltpu.make_async_copy(ones_v, a_hbm.at[idx_v.at[j]], sem).wait()
        return carry

    lax.fori_loop(0, _CH, drain, 0)


_BM = 64


def _make_layer(relu):
    def body(a_ref, x_ref, wt_ref, b_ref, o_ref):
        a = a_ref[...]
        acc = jnp.dot(a, x_ref[...], preferred_element_type=jnp.float32)
        deg = jnp.maximum(jnp.sum(a, axis=1, keepdims=True), 0.5)
        y = jnp.dot(acc / deg, wt_ref[...], preferred_element_type=jnp.float32)
        y = y + b_ref[...]
        if relu:
            y = jnp.maximum(y, 0.0)
        o_ref[...] = y

    return pl.pallas_call(
        body,
        grid=(_NP // _BM,),
        in_specs=[
            pl.BlockSpec((_BM, _NP), lambda i: (i, 0)),
            pl.BlockSpec((_NP, _D), lambda i: (0, 0)),
            pl.BlockSpec((_D, _D), lambda i: (0, 0)),
            pl.BlockSpec((1, _D), lambda i: (0, 0)),
        ],
        out_specs=pl.BlockSpec((_BM, _D), lambda i: (i, 0)),
        out_shape=jax.ShapeDtypeStruct((_NP, _D), jnp.float32),
    )


_layer_relu = _make_layer(True)
_layer_lin = _make_layer(False)


def kernel(edges, graph_embedding, W1, b1, W2, b2):
    src = edges[:, 0]
    dst = edges[:, 1]
    ar = jnp.arange(_N, dtype=jnp.int32)
    pad = _TOT - (2 * _E + _N)
    zpad = jnp.zeros((pad,), jnp.int32)
    s_all = jnp.concatenate([src, dst, ar, zpad]).reshape(_NW, _CH, 128)
    d_all = jnp.concatenate([dst, src, ar, zpad]).reshape(_NW, _CH, 128)

    a_ref = jax.new_ref(jnp.zeros((_NP * _NP,), jnp.float32))
    _scatter_adj(s_all, d_all, a_ref)
    adj = a_ref[...].reshape(_NP, _NP)

    x0 = jnp.zeros((_NP, _D), jnp.float32).at[:_N].set(graph_embedding)
    h1 = _layer_relu(adj, x0, W1.T, b1.reshape(1, _D))
    h2 = _layer_lin(adj, h1, W2.T, b2.reshape(1, _D))
    return h2[:_N]


# speedup vs baseline: 6.6873x; 1.0006x over previous
import functools

import jax
import jax.numpy as jnp
from jax import lax
from jax.experimental import pallas as pl
from jax.experimental.pallas import tpu as pltpu
from jax.experimental.pallas import tpu_sc as plsc

_N = 10000
_D = 128
_E = 320000
_NP = 10240

_NW = 32
_CH = 159
_C = _CH * 128
_TOT = _NW * _C

_sc_mesh = plsc.VectorSubcoreMesh(core_axis_name="c", subcore_axis_name="s")


@functools.partial(
    pl.kernel,
    out_type=(),
    mesh=_sc_mesh,
    scratch_types=[
        pltpu.VMEM((_CH, 128), jnp.int32),
        pltpu.VMEM((_CH, 128), jnp.int32),
        pltpu.VMEM((_CH, 128), jnp.int32),
        pltpu.VMEM((128,), jnp.float32),
        pltpu.SemaphoreType.DMA,
    ],
)
def _scatter_adj(src_hbm, dst_hbm, a_hbm, src_v, dst_v, idx_v, ones_v, sem):
    w = lax.axis_index("s") * 2 + lax.axis_index("c")
    pltpu.sync_copy(src_hbm.at[w], src_v)
    pltpu.sync_copy(dst_hbm.at[w], dst_v)
    for k in range(8):
        ones_v[pl.ds(16 * k, 16)] = jnp.ones((16,), jnp.float32)

    def jbody(j, carry):
        for k in range(8):
            sl = pl.ds(16 * k, 16)
            idx_v[j, sl] = src_v[j, sl] * _NP + dst_v[j, sl]
        return carry

    lax.fori_loop(0, _CH, jbody, 0)

    def fire(j, carry):
        pltpu.async_copy(ones_v, a_hbm.at[idx_v.at[j]], sem)
        return carry

    lax.fori_loop(0, _CH, fire, 0, unroll=8)

    pltpu.make_async_copy(src_hbm.at[w], idx_v, sem).wait()


_BM = 64


def _make_layer(relu):
    def body(a_ref, x_ref, wt_ref, b_ref, o_ref):
        a = a_ref[...]
        acc = jnp.dot(a, x_ref[...], preferred_element_type=jnp.float32)
        deg = jnp.maximum(jnp.sum(a, axis=1, keepdims=True), 0.5)
        y = jnp.dot(acc / deg, wt_ref[...], preferred_element_type=jnp.float32)
        y = y + b_ref[...]
        if relu:
            y = jnp.maximum(y, 0.0)
        o_ref[...] = y

    return pl.pallas_call(
        body,
        grid=(_NP // _BM,),
        in_specs=[
            pl.BlockSpec((_BM, _NP), lambda i: (i, 0)),
            pl.BlockSpec((_NP, _D), lambda i: (0, 0)),
            pl.BlockSpec((_D, _D), lambda i: (0, 0)),
            pl.BlockSpec((1, _D), lambda i: (0, 0)),
        ],
        out_specs=pl.BlockSpec((_BM, _D), lambda i: (i, 0)),
        out_shape=jax.ShapeDtypeStruct((_NP, _D), jnp.float32),
    )


_layer_relu = _make_layer(True)
_layer_lin = _make_layer(False)


def kernel(edges, graph_embedding, W1, b1, W2, b2):
    src = edges[:, 0]
    dst = edges[:, 1]
    ar = jnp.arange(_N, dtype=jnp.int32)
    pad = _TOT - (2 * _E + _N)
    zpad = jnp.zeros((pad,), jnp.int32)
    s_all = jnp.concatenate([src, dst, ar, zpad]).reshape(_NW, _CH, 128)
    d_all = jnp.concatenate([dst, src, ar, zpad]).reshape(_NW, _CH, 128)

    a_ref = jax.new_ref(jnp.zeros((_NP * _NP,), jnp.float32))
    _scatter_adj(s_all, d_all, a_ref)
    adj = a_ref[...].reshape(_NP, _NP)

    x0 = jnp.zeros((_NP, _D), jnp.float32).at[:_N].set(graph_embedding)
    h1 = _layer_relu(adj, x0, W1.T, b1.reshape(1, _D))
    h2 = _layer_lin(adj, h1, W2.T, b2.reshape(1, _D))
    return h2[:_N]


# speedup vs baseline: 6.7042x; 1.0025x over previous
import functools

import jax
import jax.numpy as jnp
from jax import lax
from jax.experimental import pallas as pl
from jax.experimental.pallas import tpu as pltpu
from jax.experimental.pallas import tpu_sc as plsc

_N = 10000
_D = 128
_E = 320000
_NP = 10240

_NW = 32
_CH = 159
_C = _CH * 128
_TOT = _NW * _C

_sc_mesh = plsc.VectorSubcoreMesh(core_axis_name="c", subcore_axis_name="s")


@functools.partial(
    pl.kernel,
    out_type=(),
    mesh=_sc_mesh,
    scratch_types=[
        pltpu.VMEM((_C,), jnp.int32),
        pltpu.VMEM((_C,), jnp.int32),
        pltpu.VMEM((_C,), jnp.int32),
        pltpu.VMEM((_C,), jnp.float32),
        pltpu.SemaphoreType.DMA,
    ],
)
def _scatter_adj(src_hbm, dst_hbm, a_hbm, src_v, dst_v, idx_v, ones_v, sem):
    w = lax.axis_index("s") * 2 + lax.axis_index("c")
    pltpu.sync_copy(src_hbm.at[w], src_v)
    pltpu.sync_copy(dst_hbm.at[w], dst_v)

    def jbody(j, carry):
        sl = pl.ds(16 * j, 16)
        idx_v[sl] = src_v[sl] * _NP + dst_v[sl]
        ones_v[sl] = jnp.ones((16,), jnp.float32)
        return carry

    lax.fori_loop(0, _C // 16, jbody, 0, unroll=8)

    pltpu.async_copy(ones_v, a_hbm.at[idx_v], sem).wait()


_BM = 64


def _make_layer(relu):
    def body(a_ref, x_ref, wt_ref, b_ref, o_ref):
        a = a_ref[...]
        acc = jnp.dot(a, x_ref[...], preferred_element_type=jnp.float32)
        deg = jnp.maximum(jnp.sum(a, axis=1, keepdims=True), 0.5)
        y = jnp.dot(acc / deg, wt_ref[...], preferred_element_type=jnp.float32)
        y = y + b_ref[...]
        if relu:
            y = jnp.maximum(y, 0.0)
        o_ref[...] = y

    return pl.pallas_call(
        body,
        grid=(_NP // _BM,),
        in_specs=[
            pl.BlockSpec((_BM, _NP), lambda i: (i, 0)),
            pl.BlockSpec((_NP, _D), lambda i: (0, 0)),
            pl.BlockSpec((_D, _D), lambda i: (0, 0)),
            pl.BlockSpec((1, _D), lambda i: (0, 0)),
        ],
        out_specs=pl.BlockSpec((_BM, _D), lambda i: (i, 0)),
        out_shape=jax.ShapeDtypeStruct((_NP, _D), jnp.float32),
    )


_layer_relu = _make_layer(True)
_layer_lin = _make_layer(False)


def kernel(edges, graph_embedding, W1, b1, W2, b2):
    src = edges[:, 0]
    dst = edges[:, 1]
    ar = jnp.arange(_N, dtype=jnp.int32)
    pad = _TOT - (2 * _E + _N)
    zpad = jnp.zeros((pad,), jnp.int32)
    s_all = jnp.concatenate([src, dst, ar, zpad]).reshape(_NW, _C)
    d_all = jnp.concatenate([dst, src, ar, zpad]).reshape(_NW, _C)

    a_ref = jax.new_ref(jnp.zeros((_NP * _NP,), jnp.float32))
    _scatter_adj(s_all, d_all, a_ref)
    adj = a_ref[...].reshape(_NP, _NP)

    x0 = jnp.zeros((_NP, _D), jnp.float32).at[:_N].set(graph_embedding)
    h1 = _layer_relu(adj, x0, W1.T, b1.reshape(1, _D))
    h2 = _layer_lin(adj, h1, W2.T, b2.reshape(1, _D))
    return h2[:_N]


# speedup vs baseline: 7.2483x; 1.0812x over previous
import functools

import jax
import jax.numpy as jnp
from jax import lax
from jax.experimental import pallas as pl
from jax.experimental.pallas import tpu as pltpu
from jax.experimental.pallas import tpu_sc as plsc

_N = 10000
_D = 128
_E = 320000
_NP = 10112

_NW = 32
_CH = 159
_C = _CH * 128
_TOT = _NW * _C

_sc_mesh = plsc.VectorSubcoreMesh(core_axis_name="c", subcore_axis_name="s")


@functools.partial(
    pl.kernel,
    out_type=(),
    mesh=_sc_mesh,
    scratch_types=[
        pltpu.VMEM((_C,), jnp.int32),
        pltpu.VMEM((_C,), jnp.int32),
        pltpu.VMEM((_C,), jnp.int32),
        pltpu.VMEM((_C,), jnp.float32),
        pltpu.SemaphoreType.DMA,
    ],
)
def _scatter_adj(src_hbm, dst_hbm, a_hbm, src_v, dst_v, idx_v, ones_v, sem):
    w = lax.axis_index("s") * 2 + lax.axis_index("c")
    pltpu.sync_copy(src_hbm.at[w], src_v)
    pltpu.sync_copy(dst_hbm.at[w], dst_v)

    def jbody(j, carry):
        sl = pl.ds(16 * j, 16)
        idx_v[sl] = src_v[sl] * _NP + dst_v[sl]
        ones_v[sl] = jnp.ones((16,), jnp.float32)
        return carry

    lax.fori_loop(0, _C // 16, jbody, 0, unroll=8)

    pltpu.async_copy(ones_v, a_hbm.at[idx_v], sem).wait()


_BM = 128


def _make_layer(relu):
    def body(a_ref, x_ref, wt_ref, b_ref, o_ref):
        a = a_ref[...]
        acc = jnp.dot(a, x_ref[...], preferred_element_type=jnp.float32)
        deg = jnp.maximum(jnp.sum(a, axis=1, keepdims=True), 0.5)
        y = jnp.dot(acc / deg, wt_ref[...], preferred_element_type=jnp.float32)
        y = y + b_ref[...]
        if relu:
            y = jnp.maximum(y, 0.0)
        o_ref[...] = y

    return pl.pallas_call(
        body,
        grid=(_NP // _BM,),
        in_specs=[
            pl.BlockSpec((_BM, _NP), lambda i: (i, 0)),
            pl.BlockSpec((_NP, _D), lambda i: (0, 0)),
            pl.BlockSpec((_D, _D), lambda i: (0, 0)),
            pl.BlockSpec((1, _D), lambda i: (0, 0)),
        ],
        out_specs=pl.BlockSpec((_BM, _D), lambda i: (i, 0)),
        out_shape=jax.ShapeDtypeStruct((_NP, _D), jnp.float32),
    )


_layer_relu = _make_layer(True)
_layer_lin = _make_layer(False)


def kernel(edges, graph_embedding, W1, b1, W2, b2):
    src = edges[:, 0]
    dst = edges[:, 1]
    ar = jnp.arange(_N, dtype=jnp.int32)
    pad = _TOT - (2 * _E + _N)
    zpad = jnp.zeros((pad,), jnp.int32)
    s_all = jnp.concatenate([src, dst, ar, zpad]).reshape(_NW, _C)
    d_all = jnp.concatenate([dst, src, ar, zpad]).reshape(_NW, _C)

    a_ref = jax.new_ref(jnp.zeros((_NP * _NP,), jnp.float32))
    _scatter_adj(s_all, d_all, a_ref)
    adj = a_ref[...].reshape(_NP, _NP)

    x0 = jnp.zeros((_NP, _D), jnp.float32).at[:_N].set(graph_embedding)
    h1 = _layer_relu(adj, x0, W1.T, b1.reshape(1, _D))
    h2 = _layer_lin(adj, h1, W2.T, b2.reshape(1, _D))
    return h2[:_N]


# speedup vs baseline: 7.5099x; 1.0361x over previous
import functools

import jax
import jax.numpy as jnp
from jax import lax
from jax.experimental import pallas as pl
from jax.experimental.pallas import tpu as pltpu
from jax.experimental.pallas import tpu_sc as plsc

_N = 10000
_D = 128
_E = 320000
_NP = 10240
_KT = _NP // _D

_NW = 32
_CH = 159
_C = _CH * 128
_TOT = _NW * _C

_sc_mesh = plsc.VectorSubcoreMesh(core_axis_name="c", subcore_axis_name="s")


@functools.partial(
    pl.kernel,
    out_type=(),
    mesh=_sc_mesh,
    scratch_types=[
        pltpu.VMEM((_C,), jnp.int32),
        pltpu.VMEM((_C,), jnp.int32),
        pltpu.VMEM((_C,), jnp.int32),
        pltpu.VMEM((_C,), jnp.float32),
        pltpu.SemaphoreType.DMA,
    ],
)
def _scatter_adj(src_hbm, dst_hbm, a_hbm, src_v, dst_v, idx_v, ones_v, sem):
    w = lax.axis_index("s") * 2 + lax.axis_index("c")
    pltpu.sync_copy(src_hbm.at[w], src_v)
    pltpu.sync_copy(dst_hbm.at[w], dst_v)

    def jbody(j, carry):
        sl = pl.ds(16 * j, 16)
        idx_v[sl] = src_v[sl] * _NP + dst_v[sl]
        ones_v[sl] = jnp.ones((16,), jnp.float32)
        return carry

    lax.fori_loop(0, _C // 16, jbody, 0, unroll=8)

    pltpu.async_copy(ones_v, a_hbm.at[idx_v], sem).wait()


_BM = 256


def _make_layer(relu):
    def body(a_ref, x_ref, wt_ref, b_ref, o_ref):
        acc = jnp.zeros((_BM, _D), jnp.float32)
        deg = jnp.zeros((_BM, 1), jnp.float32)
        for k in range(_KT):
            ak = a_ref[:, k, :]
            acc += jnp.dot(ak, x_ref[k], preferred_element_type=jnp.float32)
            deg += jnp.sum(ak, axis=1, keepdims=True)
        deg = jnp.maximum(deg, 0.5)
        y = jnp.dot(acc / deg, wt_ref[...], preferred_element_type=jnp.float32)
        y = y + b_ref[...]
        if relu:
            y = jnp.maximum(y, 0.0)
        o_ref[...] = y

    return pl.pallas_call(
        body,
        grid=(_NP // _BM,),
        in_specs=[
            pl.BlockSpec((_BM, _KT, _D), lambda i: (i, 0, 0)),
            pl.BlockSpec((_KT, _D, _D), lambda i: (0, 0, 0)),
            pl.BlockSpec((_D, _D), lambda i: (0, 0)),
            pl.BlockSpec((1, _D), lambda i: (0, 0)),
        ],
        out_specs=pl.BlockSpec((_BM, _D), lambda i: (i, 0)),
        out_shape=jax.ShapeDtypeStruct((_NP, _D), jnp.float32),
    )


_layer_relu = _make_layer(True)
_layer_lin = _make_layer(False)


def kernel(edges, graph_embedding, W1, b1, W2, b2):
    src = edges[:, 0]
    dst = edges[:, 1]
    ar = jnp.arange(_N, dtype=jnp.int32)
    pad = _TOT - (2 * _E + _N)
    zpad = jnp.zeros((pad,), jnp.int32)
    s_all = jnp.concatenate([src, dst, ar, zpad]).reshape(_NW, _C)
    d_all = jnp.concatenate([dst, src, ar, zpad]).reshape(_NW, _C)

    a_ref = jax.new_ref(jnp.zeros((_NP * _NP,), jnp.float32))
    _scatter_adj(s_all, d_all, a_ref)
    adj3 = a_ref[...].reshape(_NP, _KT, _D)

    x0 = jnp.zeros((_NP, _D), jnp.float32).at[:_N].set(graph_embedding)
    x3 = x0.reshape(_KT, _D, _D)
    h1 = _layer_relu(adj3, x3, W1.T, b1.reshape(1, _D))
    h2 = _layer_lin(adj3, h1.reshape(_KT, _D, _D), W2.T, b2.reshape(1, _D))
    return h2[:_N]
